# baseline (speedup 1.0000x reference)
import jax
import jax.numpy as jnp
from jax.experimental import pallas as pl
from jax.experimental.pallas import tpu as pltpu

_NSPLIT = 8


def kernel(x, emb_weight):
    sl = x.shape[1]
    dim = emb_weight.shape[1]
    rows = sl // _NSPLIT

    def body(in_hbm, out_hbm, sems):
        copies = [
            pltpu.make_async_copy(
                in_hbm.at[pl.ds(i * rows, rows)],
                out_hbm.at[pl.ds(i * rows, rows)],
                sems.at[i],
            )
            for i in range(_NSPLIT)
        ]
        for c in copies:
            c.start()
        for c in copies:
            c.wait()

    return pl.pallas_call(
        body,
        out_shape=jax.ShapeDtypeStruct((sl, dim), emb_weight.dtype),
        in_specs=[pl.BlockSpec(memory_space=pl.ANY)],
        out_specs=pl.BlockSpec(memory_space=pl.ANY),
        scratch_shapes=[pltpu.SemaphoreType.DMA((_NSPLIT,))],
    )(emb_weight)


# speedup vs baseline: 42.9592x; 42.9592x over previous
import jax
import jax.numpy as jnp
from jax.experimental import pallas as pl

_N_BLOCKS = 8


def _copy_body(in_ref, out_ref):
    out_ref[...] = in_ref[...]


def kernel(x, emb_weight):
    sl = x.shape[1]
    rows, dim = emb_weight.shape
    del rows
    block_rows = sl // _N_BLOCKS
    return pl.pallas_call(
        _copy_body,
        out_shape=jax.ShapeDtypeStruct((sl, dim), emb_weight.dtype),
        grid=(_N_BLOCKS,),
        in_specs=[pl.BlockSpec((block_rows, dim), lambda i: (i, 0))],
        out_specs=pl.BlockSpec((block_rows, dim), lambda i: (i, 0)),
    )(emb_weight)


# speedup vs baseline: 45.6939x; 1.0637x over previous
import jax
import jax.numpy as jnp
from jax.experimental import pallas as pl

_N_BLOCKS = 4


def _copy_body(in_ref, out_ref):
    out_ref[...] = in_ref[...]


def kernel(x, emb_weight):
    sl = x.shape[1]
    rows, dim = emb_weight.shape
    del rows
    block_rows = sl // _N_BLOCKS
    return pl.pallas_call(
        _copy_body,
        out_shape=jax.ShapeDtypeStruct((sl, dim), emb_weight.dtype),
        grid=(_N_BLOCKS,),
        in_specs=[pl.BlockSpec((block_rows, dim), lambda i: (i, 0))],
        out_specs=pl.BlockSpec((block_rows, dim), lambda i: (i, 0)),
    )(emb_weight)
